# baseline (speedup 1.0000x reference)
import functools

import jax
import jax.numpy as jnp
from jax import lax
from jax.experimental import pallas as pl
from jax.experimental.pallas import tpu as pltpu
from jax.experimental.pallas import tpu_sc as plsc

B = 16384
F = 512
G = 8
P = F // G
L = 16
NC = 2
NS = 16
NW = NC * NS

ROWS_PER_W = B // NW
CH = 64
NCHUNK = ROWS_PER_W // CH

_mesh = plsc.VectorSubcoreMesh(core_axis_name="c", subcore_axis_name="s")


@functools.partial(
    pl.kernel,
    out_type=jax.ShapeDtypeStruct((P, B), jnp.float32),
    mesh=_mesh,
    scratch_types=[
        pltpu.VMEM((CH, F + 8), jnp.float32),
        pltpu.VMEM((CH, F + 8), jnp.float32),
        pltpu.VMEM((P, 2 * CH), jnp.float32),
        pltpu.VMEM((P, 2 * CH), jnp.float32),
        pltpu.SemaphoreType.DMA,
        pltpu.SemaphoreType.DMA,
        pltpu.SemaphoreType.DMA,
        pltpu.SemaphoreType.DMA,
    ],
    compiler_params=pltpu.CompilerParams(needs_layout_passes=False),
)
def _pool_sc(x_hbm, out_hbm, in0, in1, o0, o1, si0, si1, so0, so1):
    wid = lax.axis_index("s") * NC + lax.axis_index("c")
    row0 = wid * ROWS_PER_W

    ins = (in0, in1)
    outs = (o0, o1)
    isems = (si0, si1)
    osems = (so0, so1)

    lane = lax.iota(jnp.int32, L)

    in_copies = [None, None]
    out_copies = [None, None]
    in_copies[0] = pltpu.async_copy(
        x_hbm.at[pl.ds(row0, CH)], ins[0].at[:, pl.ds(0, F)], isems[0]
    )

    for c in range(NCHUNK):
        cur = c % 2
        if c + 1 < NCHUNK:
            nxt = (c + 1) % 2
            in_copies[nxt] = pltpu.async_copy(
                x_hbm.at[pl.ds(row0 + (c + 1) * CH, CH)],
                ins[nxt].at[:, pl.ds(0, F)],
                isems[nxt],
            )
        in_copies[cur].wait()
        ob = (c // 2) % 2
        half = c % 2
        if half == 0 and out_copies[ob] is not None:
            out_copies[ob].wait()

        in_ref = ins[cur]
        out_ref = outs[ob]

        @plsc.parallel_loop(0, P, step=1, unroll=2)
        def _body(i):
            for q in range(CH // L):
                row_idx = lane + q * L
                acc = plsc.load_gather(
                    in_ref, [row_idx, jnp.full((L,), i * G, jnp.int32)]
                )
                for k in range(1, G):
                    acc = acc + plsc.load_gather(
                        in_ref, [row_idx, jnp.full((L,), i * G + k, jnp.int32)]
                    )
                out_ref[i, pl.ds(half * CH + q * L, L)] = acc * (1.0 / G)

        if half == 1:
            out_copies[ob] = pltpu.async_copy(
                out_ref,
                out_hbm.at[:, pl.ds(row0 + (c - 1) * CH, 2 * CH)],
                osems[ob],
            )

    out_copies[0].wait()
    out_copies[1].wait()


def kernel(gene_set_features):
    return _pool_sc(gene_set_features).T


# speedup vs baseline: 2.6979x; 2.6979x over previous
import functools

import jax
import jax.numpy as jnp
from jax import lax
from jax.experimental import pallas as pl
from jax.experimental.pallas import tpu as pltpu
from jax.experimental.pallas import tpu_sc as plsc

B = 16384
F = 512
G = 8
P = F // G
L = 16
NC = 2
NS = 16
NW = NC * NS

ROWS_PER_W = B // NW
CH = 64
NCHUNK = ROWS_PER_W // CH
NG = P // L

_mesh = plsc.VectorSubcoreMesh(core_axis_name="c", subcore_axis_name="s")


@functools.partial(
    pl.kernel,
    out_type=jax.ShapeDtypeStruct((P, B), jnp.float32),
    mesh=_mesh,
    scratch_types=[
        pltpu.VMEM((CH, F), jnp.float32),
        pltpu.VMEM((CH, F), jnp.float32),
        pltpu.VMEM((NG, CH, L), jnp.float32),
        pltpu.VMEM((P, 2 * CH), jnp.float32),
        pltpu.VMEM((P, 2 * CH), jnp.float32),
        pltpu.SemaphoreType.DMA,
        pltpu.SemaphoreType.DMA,
        pltpu.SemaphoreType.DMA,
        pltpu.SemaphoreType.DMA,
    ],
    compiler_params=pltpu.CompilerParams(needs_layout_passes=False),
)
def _pool_sc(x_hbm, out_hbm, in0, in1, stage, o0, o1, si0, si1, so0, so1):
    wid = lax.axis_index("s") * NC + lax.axis_index("c")
    row0 = wid * ROWS_PER_W

    ins = (in0, in1)
    outs = (o0, o1)
    isems = (si0, si1)
    osems = (so0, so1)

    lane = lax.iota(jnp.int32, L)
    lane8 = lane * G

    in_copies = [None, None]
    out_copies = [None, None]
    in_copies[0] = pltpu.async_copy(
        x_hbm.at[pl.ds(row0, CH)], ins[0], isems[0]
    )

    for c in range(NCHUNK):
        cur = c % 2
        if c + 1 < NCHUNK:
            nxt = (c + 1) % 2
            in_copies[nxt] = pltpu.async_copy(
                x_hbm.at[pl.ds(row0 + (c + 1) * CH, CH)],
                ins[nxt],
                isems[nxt],
            )
        in_copies[cur].wait()
        ob = (c // 2) % 2
        half = c % 2
        if half == 0 and out_copies[ob] is not None:
            out_copies[ob].wait()

        in_ref = ins[cur]
        out_ref = outs[ob]

        @plsc.parallel_loop(0, CH, step=1, unroll=2)
        def _sums(r):
            row_idx = jnp.full((L,), r, jnp.int32)
            for g in range(NG):
                col0 = lane8 + g * (L * G)
                acc = plsc.load_gather(in_ref, [row_idx, col0])
                for k in range(1, G):
                    acc = acc + plsc.load_gather(in_ref, [row_idx, col0 + k])
                stage[g, r, pl.ds(0, L)] = acc * (1.0 / G)

        @plsc.parallel_loop(0, P, step=1, unroll=2)
        def _tr(i):
            g = i // L
            col = i % L
            gv = jnp.full((L,), g, jnp.int32)
            cv = jnp.full((L,), col, jnp.int32)
            for q in range(CH // L):
                v = plsc.load_gather(stage, [gv, lane + q * L, cv])
                out_ref[i, pl.ds(half * CH + q * L, L)] = v

        if half == 1:
            out_copies[ob] = pltpu.async_copy(
                out_ref,
                out_hbm.at[:, pl.ds(row0 + (c - 1) * CH, 2 * CH)],
                osems[ob],
            )

    out_copies[0].wait()
    out_copies[1].wait()


def kernel(gene_set_features):
    return _pool_sc(gene_set_features).T
